# baseline (speedup 1.0000x reference)
import functools

import jax
import jax.numpy as jnp
from jax import lax
from jax.experimental import pallas as pl
from jax.experimental.pallas import tpu as pltpu
from jax.experimental.pallas import tpu_sc as plsc

BN_EPS = 1e-5
COS_EPS = 1e-8
LRELU_SLOPE = 0.01


def _pack_bf16_pair(lo_f32, hi_f32):
    lo_b = lax.bitcast_convert_type(
        lo_f32.astype(jnp.bfloat16).astype(jnp.float32), jnp.int32)
    hi_b = lax.bitcast_convert_type(
        hi_f32.astype(jnp.bfloat16).astype(jnp.float32), jnp.int32)
    return lax.shift_right_logical(lo_b, 16) | ((hi_b >> 16) << 16)


def _unpack_bf16_pair(w32):
    lo = lax.bitcast_convert_type(w32 << 16, jnp.float32)
    hi = lax.bitcast_convert_type((w32 >> 16) << 16, jnp.float32)
    return jnp.concatenate([lo, hi], axis=-1)


def _p1_body(x_ref, w_ref, h_ref, sr_ref):
    xb = x_ref[...]
    hb = lax.dot_general(xb, w_ref[...], (((1,), (1,)), ((), ())),
                         preferred_element_type=jnp.float32)
    half = hb.shape[1] // 2
    h_ref[...] = _pack_bf16_pair(hb[:, :half], hb[:, half:])
    sr = jnp.concatenate(
        [jnp.sum(hb, axis=0, keepdims=True),
         jnp.sum(hb * hb, axis=0, keepdims=True)], axis=0)

    @pl.when(pl.program_id(0) == 0)
    def _():
        sr_ref[...] = sr

    @pl.when(pl.program_id(0) != 0)
    def _():
        sr_ref[...] += sr


def _encode_and_stats(x, w_conv):
    m, k = x.shape
    ls = w_conv.shape[0]
    bm = 8192
    return pl.pallas_call(
        _p1_body,
        grid=(m // bm,),
        in_specs=[
            pl.BlockSpec((bm, k), lambda i: (i, 0)),
            pl.BlockSpec((ls, k), lambda i: (0, 0)),
        ],
        out_specs=[
            pl.BlockSpec((bm, ls // 2), lambda i: (i, 0)),
            pl.BlockSpec((2, ls), lambda i: (0, 0)),
        ],
        out_shape=[
            jax.ShapeDtypeStruct((m, ls // 2), jnp.int32),
            jax.ShapeDtypeStruct((2, ls), jnp.float32),
        ],
        compiler_params=pltpu.CompilerParams(
            dimension_semantics=("arbitrary",)),
    )(x, w_conv)


def _sc_gather(hi, idx3):
    info = plsc.get_sparse_core_info()
    nc, ns = info.num_cores, info.num_subcores
    nw = nc * ns
    m, wd = hi.shape
    rpw = m // nw
    nch = idx3.shape[1]
    mesh = plsc.VectorSubcoreMesh(core_axis_name="c", subcore_axis_name="s")

    @functools.partial(
        pl.kernel,
        mesh=mesh,
        out_type=jax.ShapeDtypeStruct((m, wd), jnp.int32),
        scratch_types=[
            pltpu.VMEM((nch, 128), jnp.int32),
            pltpu.VMEM((rpw, wd), jnp.int32),
            pltpu.SemaphoreType.DMA,
        ],
        compiler_params=pltpu.CompilerParams(use_tc_tiling_on_sc=False),
    )
    def k(h_hbm, idx_hbm, out_hbm, idx_v, rows_v, sem):
        wid = lax.axis_index("s") * nc + lax.axis_index("c")
        base = wid * rpw
        pltpu.sync_copy(idx_hbm.at[wid], idx_v)
        copies = []
        for j in range(nch):
            copies.append(pltpu.async_copy(
                h_hbm.at[idx_v.at[j]], rows_v.at[pl.ds(j * 128, 128)], sem))
        for c in copies:
            c.wait()
        pltpu.sync_copy(rows_v, out_hbm.at[pl.ds(base, rpw)])

    return k(hi, idx3)


def _lane_roll(x):
    return jnp.concatenate([x[:, 1:], x[:, :1]], axis=1)


def _p2_body(n_rows, h_ref, hp_ref, sr_ref, gr_ref, br_ref, w_ref, blt_ref,
             m_ref, o0_ref, o1_ref, l_ref):
    w = w_ref[...]
    sr = sr_ref[...]
    mean_r = sr[0:1, :] / n_rows
    var_r = sr[1:2, :] / n_rows - mean_r * mean_r
    scale_r = gr_ref[...] * lax.rsqrt(var_r + BN_EPS)
    shift_r = br_ref[...] - mean_r * scale_r
    blt = blt_ref[...]

    def head(hb_packed):
        a = _unpack_bf16_pair(hb_packed) * scale_r + shift_r
        a = jnp.where(a >= 0, a, LRELU_SLOPE * a)
        return lax.dot_general(w, a, (((1,), (1,)), ((), ())),
                               preferred_element_type=jnp.float32) + blt

    z = head(h_ref[0])
    zp = head(hp_ref[0])

    zn = _lane_roll(z)
    r = 1.0 / jnp.maximum(
        jnp.sqrt(jnp.sum(z * z, axis=0, keepdims=True)), COS_EPS)
    rp = 1.0 / jnp.maximum(
        jnp.sqrt(jnp.sum(zp * zp, axis=0, keepdims=True)), COS_EPS)
    rn = _lane_roll(r)

    pos = jnp.sum(z * zn, axis=0, keepdims=True) * (r * rn)
    neg = jnp.sum(z * zp, axis=0, keepdims=True) * (r * rp)

    mx = jnp.maximum(pos, neg)
    lse = mx + jnp.log(jnp.exp(pos - mx) + jnp.exp(neg - mx))
    o0 = pos - lse
    o0_ref[...] = o0[None]
    o1_ref[...] = (neg - lse)[None]
    l_ref[...] = (-o0 * (1.0 - m_ref[0]))[None]


def _score(h3, hp3, sr, gamma, beta, w_lin, b_lin, mask3, b, t):
    ls = w_lin.shape[0]
    n_rows = float(b * t)
    out_spec = pl.BlockSpec((1, 1, t), lambda bi: (bi, 0, 0))
    full3 = lambda bi: (bi, 0, 0)
    const2 = lambda bi: (0, 0)
    return pl.pallas_call(
        functools.partial(_p2_body, n_rows),
        grid=(b,),
        in_specs=[
            pl.BlockSpec((1, t, ls // 2), full3),
            pl.BlockSpec((1, t, ls // 2), full3),
            pl.BlockSpec((2, ls), const2),
            pl.BlockSpec((1, ls), const2),
            pl.BlockSpec((1, ls), const2),
            pl.BlockSpec((ls, ls), const2),
            pl.BlockSpec((ls, 1), const2),
            pl.BlockSpec((1, 1, t), full3),
        ],
        out_specs=[out_spec, out_spec, out_spec],
        out_shape=[jax.ShapeDtypeStruct((b, 1, t), jnp.float32)] * 3,
        compiler_params=pltpu.CompilerParams(
            dimension_semantics=("parallel",)),
    )(h3, hp3, sr, gamma.reshape(1, ls), beta.reshape(1, ls), w_lin,
      b_lin.reshape(ls, 1), mask3)


def kernel(logits, padding_mask, W_conv, gamma, beta, W_lin, b_lin):
    b, t, i_dim = logits.shape
    ls = W_conv.shape[0]
    bh = b // 2
    mh = bh * t

    x1 = logits[:bh].reshape(mh, i_dim)
    x2 = logits[bh:].reshape(mh, i_dim)
    h1, s1 = _encode_and_stats(x1, W_conv)
    h2, s2 = _encode_and_stats(x2, W_conv)

    with jax.ensure_compile_time_eval():
        perm = jax.random.permutation(
            jax.random.fold_in(jax.random.key(42), 0), t - 1)
        perm_full = jnp.concatenate(
            [perm.astype(jnp.int32), jnp.array([t - 1], jnp.int32)])
        idx = (jnp.arange(bh, dtype=jnp.int32)[:, None] * t
               + perm_full[None, :]).reshape(-1)
        idx3 = idx.reshape(32, -1, 128)

    hp1 = _sc_gather(h1, idx3)
    hp2 = _sc_gather(h2, idx3)
    sr = s1 + s2

    maskf = padding_mask.astype(jnp.float32).reshape(b, 1, t)
    o0a, o1a, la = _score(
        h1.reshape(bh, t, ls // 2), hp1.reshape(bh, t, ls // 2), sr,
        gamma, beta, W_lin, b_lin, maskf[:bh], bh, t)
    o0b, o1b, lb = _score(
        h2.reshape(bh, t, ls // 2), hp2.reshape(bh, t, ls // 2), sr,
        gamma, beta, W_lin, b_lin, maskf[bh:], bh, t)

    out0 = jnp.concatenate([o0a, o0b], axis=0).reshape(b, t)
    out1 = jnp.concatenate([o1a, o1b], axis=0).reshape(b, t)
    loss = jnp.concatenate([la, lb], axis=0).reshape(b, t)
    out = jnp.stack([out0[:, :t - 1], out1[:, :t - 1]], axis=-1)
    return (out, loss[:, :t - 1])


# speedup vs baseline: 1.3197x; 1.3197x over previous
import functools

import jax
import jax.numpy as jnp
from jax import lax
from jax.experimental import pallas as pl
from jax.experimental.pallas import tpu as pltpu
from jax.experimental.pallas import tpu_sc as plsc

BN_EPS = 1e-5
COS_EPS = 1e-8
LRELU_SLOPE = 0.01


def _pack_bf16_pair(lo_f32, hi_f32):
    lo_b = lax.bitcast_convert_type(
        lo_f32.astype(jnp.bfloat16).astype(jnp.float32), jnp.int32)
    hi_b = lax.bitcast_convert_type(
        hi_f32.astype(jnp.bfloat16).astype(jnp.float32), jnp.int32)
    return lax.shift_right_logical(lo_b, 16) | ((hi_b >> 16) << 16)


def _unpack_bf16_pair(w32):
    lo = lax.bitcast_convert_type(w32 << 16, jnp.float32)
    hi = lax.bitcast_convert_type((w32 >> 16) << 16, jnp.float32)
    return jnp.concatenate([lo, hi], axis=-1)


def _p1_body(x_ref, w_ref, h_ref, sr_ref):
    xb = x_ref[...]
    hb = lax.dot_general(xb, w_ref[...], (((1,), (1,)), ((), ())),
                         preferred_element_type=jnp.float32)
    half = hb.shape[1] // 2
    h_ref[...] = _pack_bf16_pair(hb[:, :half], hb[:, half:])
    sr = jnp.concatenate(
        [jnp.sum(hb, axis=0, keepdims=True),
         jnp.sum(hb * hb, axis=0, keepdims=True)], axis=0)

    @pl.when(pl.program_id(0) == 0)
    def _():
        sr_ref[...] = sr

    @pl.when(pl.program_id(0) != 0)
    def _():
        sr_ref[...] += sr


def _encode_and_stats(x, w_conv):
    m, k = x.shape
    ls = w_conv.shape[0]
    bm = 8192
    return pl.pallas_call(
        _p1_body,
        grid=(m // bm,),
        in_specs=[
            pl.BlockSpec((bm, k), lambda i: (i, 0)),
            pl.BlockSpec((ls, k), lambda i: (0, 0)),
        ],
        out_specs=[
            pl.BlockSpec((bm, ls // 2), lambda i: (i, 0)),
            pl.BlockSpec((2, ls), lambda i: (0, 0)),
        ],
        out_shape=[
            jax.ShapeDtypeStruct((m, ls // 2), jnp.int32),
            jax.ShapeDtypeStruct((2, ls), jnp.float32),
        ],
        compiler_params=pltpu.CompilerParams(
            dimension_semantics=("arbitrary",)),
    )(x, w_conv)


def _sc_gather(hi, idx3):
    info = plsc.get_sparse_core_info()
    nc, ns = 1, info.num_subcores
    nw = nc * ns
    m, wd = hi.shape
    rpw = m // nw
    nch = idx3.shape[1]
    mesh = plsc.VectorSubcoreMesh(core_axis_name="c", subcore_axis_name="s",
                                  num_cores=1)

    @functools.partial(
        pl.kernel,
        mesh=mesh,
        out_type=jax.ShapeDtypeStruct((m, wd), jnp.int32),
        scratch_types=[
            pltpu.VMEM((nch, 128), jnp.int32),
            pltpu.VMEM((rpw, wd), jnp.int32),
            pltpu.SemaphoreType.DMA,
        ],
        compiler_params=pltpu.CompilerParams(use_tc_tiling_on_sc=False),
    )
    def k(h_hbm, idx_hbm, out_hbm, idx_v, rows_v, sem):
        wid = lax.axis_index("s") * nc + lax.axis_index("c")
        base = wid * rpw
        pltpu.sync_copy(idx_hbm.at[wid], idx_v)
        copies = []
        for j in range(nch):
            copies.append(pltpu.async_copy(
                h_hbm.at[idx_v.at[j]], rows_v.at[pl.ds(j * 128, 128)], sem))
        for c in copies:
            c.wait()
        pltpu.sync_copy(rows_v, out_hbm.at[pl.ds(base, rpw)])

    return k(hi, idx3)


def _lane_roll(x):
    return jnp.concatenate([x[:, 1:], x[:, :1]], axis=1)


def _p2_body(n_rows, h_ref, hp_ref, sr_ref, gr_ref, br_ref, w_ref, blt_ref,
             m_ref, o0_ref, o1_ref, l_ref):
    w = w_ref[...]
    sr = sr_ref[...]
    mean_r = sr[0:1, :] / n_rows
    var_r = sr[1:2, :] / n_rows - mean_r * mean_r
    scale_r = gr_ref[...] * lax.rsqrt(var_r + BN_EPS)
    shift_r = br_ref[...] - mean_r * scale_r
    blt = blt_ref[...]

    def head(hb_packed):
        a = _unpack_bf16_pair(hb_packed) * scale_r + shift_r
        a = jnp.where(a >= 0, a, LRELU_SLOPE * a)
        return lax.dot_general(w, a, (((1,), (1,)), ((), ())),
                               preferred_element_type=jnp.float32) + blt

    z = head(h_ref[0])
    zp = head(hp_ref[0])

    zn = _lane_roll(z)
    r = 1.0 / jnp.maximum(
        jnp.sqrt(jnp.sum(z * z, axis=0, keepdims=True)), COS_EPS)
    rp = 1.0 / jnp.maximum(
        jnp.sqrt(jnp.sum(zp * zp, axis=0, keepdims=True)), COS_EPS)
    rn = _lane_roll(r)

    pos = jnp.sum(z * zn, axis=0, keepdims=True) * (r * rn)
    neg = jnp.sum(z * zp, axis=0, keepdims=True) * (r * rp)

    mx = jnp.maximum(pos, neg)
    lse = mx + jnp.log(jnp.exp(pos - mx) + jnp.exp(neg - mx))
    o0 = pos - lse
    o0_ref[...] = o0[None]
    o1_ref[...] = (neg - lse)[None]
    l_ref[...] = (-o0 * (1.0 - m_ref[0]))[None]


def _score(h3, hp3, sr, gamma, beta, w_lin, b_lin, mask3, b, t):
    ls = w_lin.shape[0]
    n_rows = float(b * t)
    out_spec = pl.BlockSpec((1, 1, t), lambda bi: (bi, 0, 0))
    full3 = lambda bi: (bi, 0, 0)
    const2 = lambda bi: (0, 0)
    return pl.pallas_call(
        functools.partial(_p2_body, n_rows),
        grid=(b,),
        in_specs=[
            pl.BlockSpec((1, t, ls // 2), full3),
            pl.BlockSpec((1, t, ls // 2), full3),
            pl.BlockSpec((2, ls), const2),
            pl.BlockSpec((1, ls), const2),
            pl.BlockSpec((1, ls), const2),
            pl.BlockSpec((ls, ls), const2),
            pl.BlockSpec((ls, 1), const2),
            pl.BlockSpec((1, 1, t), full3),
        ],
        out_specs=[out_spec, out_spec, out_spec],
        out_shape=[jax.ShapeDtypeStruct((b, 1, t), jnp.float32)] * 3,
        compiler_params=pltpu.CompilerParams(
            dimension_semantics=("parallel",)),
    )(h3, hp3, sr, gamma.reshape(1, ls), beta.reshape(1, ls), w_lin,
      b_lin.reshape(ls, 1), mask3)


def kernel(logits, padding_mask, W_conv, gamma, beta, W_lin, b_lin):
    b, t, i_dim = logits.shape
    ls = W_conv.shape[0]
    m = b * t

    x = logits.reshape(m, i_dim)
    h, sr = _encode_and_stats(x, W_conv)

    with jax.ensure_compile_time_eval():
        perm = jax.random.permutation(
            jax.random.fold_in(jax.random.key(42), 0), t - 1)
        perm_full = jnp.concatenate(
            [perm.astype(jnp.int32), jnp.array([t - 1], jnp.int32)])
        idx = (jnp.arange(b, dtype=jnp.int32)[:, None] * t
               + perm_full[None, :]).reshape(-1)
        idx3 = idx.reshape(16, -1, 128)

    hp = _sc_gather(h, idx3)

    out0, out1, loss = _score(
        h.reshape(b, t, ls // 2), hp.reshape(b, t, ls // 2), sr, gamma, beta,
        W_lin, b_lin,
        padding_mask.astype(jnp.float32).reshape(b, 1, t), b, t)

    out = jnp.stack(
        [out0.reshape(b, t)[:, :t - 1], out1.reshape(b, t)[:, :t - 1]],
        axis=-1)
    return (out, loss.reshape(b, t)[:, :t - 1])


# speedup vs baseline: 1.3469x; 1.0206x over previous
import functools

import jax
import jax.numpy as jnp
from jax import lax
from jax.experimental import pallas as pl
from jax.experimental.pallas import tpu as pltpu
from jax.experimental.pallas import tpu_sc as plsc

BN_EPS = 1e-5
COS_EPS = 1e-8
LRELU_SLOPE = 0.01


def _pack_bf16_pair(lo_f32, hi_f32):
    lo_b = lax.bitcast_convert_type(
        lo_f32.astype(jnp.bfloat16).astype(jnp.float32), jnp.int32)
    hi_b = lax.bitcast_convert_type(
        hi_f32.astype(jnp.bfloat16).astype(jnp.float32), jnp.int32)
    return lax.shift_right_logical(lo_b, 16) | ((hi_b >> 16) << 16)


def _unpack_bf16_pair(w32):
    lo = lax.bitcast_convert_type(w32 << 16, jnp.float32)
    hi = lax.bitcast_convert_type((w32 >> 16) << 16, jnp.float32)
    return jnp.concatenate([lo, hi], axis=-1)


def _p1_body(x_ref, w_ref, h_ref, sr_ref):
    xb = x_ref[...]
    hb = lax.dot_general(xb, w_ref[...], (((1,), (1,)), ((), ())),
                         preferred_element_type=jnp.float32)
    half = hb.shape[1] // 2
    h_ref[...] = _pack_bf16_pair(hb[:, :half], hb[:, half:])
    sr = jnp.concatenate(
        [jnp.sum(hb, axis=0, keepdims=True),
         jnp.sum(hb * hb, axis=0, keepdims=True)], axis=0)

    @pl.when(pl.program_id(0) == 0)
    def _():
        sr_ref[...] = sr

    @pl.when(pl.program_id(0) != 0)
    def _():
        sr_ref[...] += sr


def _encode_and_stats(x, w_conv):
    m, k = x.shape
    ls = w_conv.shape[0]
    bm = 16384
    return pl.pallas_call(
        _p1_body,
        grid=(m // bm,),
        in_specs=[
            pl.BlockSpec((bm, k), lambda i: (i, 0)),
            pl.BlockSpec((ls, k), lambda i: (0, 0)),
        ],
        out_specs=[
            pl.BlockSpec((bm, ls // 2), lambda i: (i, 0)),
            pl.BlockSpec((2, ls), lambda i: (0, 0)),
        ],
        out_shape=[
            jax.ShapeDtypeStruct((m, ls // 2), jnp.int32),
            jax.ShapeDtypeStruct((2, ls), jnp.float32),
        ],
        compiler_params=pltpu.CompilerParams(
            dimension_semantics=("arbitrary",)),
    )(x, w_conv)


def _sc_gather(hi, idx3):
    info = plsc.get_sparse_core_info()
    nc, ns = info.num_cores, info.num_subcores
    nw = nc * ns
    m, wd = hi.shape
    rpw = m // nw
    nch = idx3.shape[1]
    mesh = plsc.VectorSubcoreMesh(core_axis_name="c", subcore_axis_name="s")

    @functools.partial(
        pl.kernel,
        mesh=mesh,
        out_type=jax.ShapeDtypeStruct((m, wd), jnp.int32),
        scratch_types=[
            pltpu.VMEM((nch, 128), jnp.int32),
            pltpu.VMEM((rpw, wd), jnp.int32),
            pltpu.SemaphoreType.DMA,
        ],
        compiler_params=pltpu.CompilerParams(use_tc_tiling_on_sc=False),
    )
    def k(h_hbm, idx_hbm, out_hbm, idx_v, rows_v, sem):
        wid = lax.axis_index("s") * nc + lax.axis_index("c")
        base = wid * rpw
        pltpu.sync_copy(idx_hbm.at[wid], idx_v)
        copies = []
        for j in range(nch):
            copies.append(pltpu.async_copy(
                h_hbm.at[idx_v.at[j]], rows_v.at[pl.ds(j * 128, 128)], sem))
        for c in copies:
            c.wait()
        pltpu.sync_copy(rows_v, out_hbm.at[pl.ds(base, rpw)])

    return k(hi, idx3)


def _lane_roll(x):
    return jnp.concatenate([x[:, 1:], x[:, :1]], axis=1)


def _p2_body(n_rows, h_ref, hp_ref, sr_ref, gr_ref, br_ref, w_ref, blt_ref,
             m_ref, o0_ref, o1_ref, l_ref):
    w = w_ref[...]
    sr = sr_ref[...]
    mean_r = sr[0:1, :] / n_rows
    var_r = sr[1:2, :] / n_rows - mean_r * mean_r
    scale_r = gr_ref[...] * lax.rsqrt(var_r + BN_EPS)
    shift_r = br_ref[...] - mean_r * scale_r
    blt = blt_ref[...]

    def head(hb_packed):
        a = _unpack_bf16_pair(hb_packed) * scale_r + shift_r
        a = jnp.where(a >= 0, a, LRELU_SLOPE * a)
        return lax.dot_general(w, a, (((1,), (1,)), ((), ())),
                               preferred_element_type=jnp.float32) + blt

    z = head(h_ref[0])
    zp = head(hp_ref[0])

    zn = _lane_roll(z)
    r = 1.0 / jnp.maximum(
        jnp.sqrt(jnp.sum(z * z, axis=0, keepdims=True)), COS_EPS)
    rp = 1.0 / jnp.maximum(
        jnp.sqrt(jnp.sum(zp * zp, axis=0, keepdims=True)), COS_EPS)
    rn = _lane_roll(r)

    pos = jnp.sum(z * zn, axis=0, keepdims=True) * (r * rn)
    neg = jnp.sum(z * zp, axis=0, keepdims=True) * (r * rp)

    mx = jnp.maximum(pos, neg)
    lse = mx + jnp.log(jnp.exp(pos - mx) + jnp.exp(neg - mx))
    o0 = pos - lse
    o0_ref[...] = o0[None]
    o1_ref[...] = (neg - lse)[None]
    l_ref[...] = (-o0 * (1.0 - m_ref[0]))[None]


def _score(h3, hp3, sr, gamma, beta, w_lin, b_lin, mask3, b, t):
    ls = w_lin.shape[0]
    n_rows = float(b * t)
    out_spec = pl.BlockSpec((1, 1, t), lambda bi: (bi, 0, 0))
    full3 = lambda bi: (bi, 0, 0)
    const2 = lambda bi: (0, 0)
    return pl.pallas_call(
        functools.partial(_p2_body, n_rows),
        grid=(b,),
        in_specs=[
            pl.BlockSpec((1, t, ls // 2), full3),
            pl.BlockSpec((1, t, ls // 2), full3),
            pl.BlockSpec((2, ls), const2),
            pl.BlockSpec((1, ls), const2),
            pl.BlockSpec((1, ls), const2),
            pl.BlockSpec((ls, ls), const2),
            pl.BlockSpec((ls, 1), const2),
            pl.BlockSpec((1, 1, t), full3),
        ],
        out_specs=[out_spec, out_spec, out_spec],
        out_shape=[jax.ShapeDtypeStruct((b, 1, t), jnp.float32)] * 3,
        compiler_params=pltpu.CompilerParams(
            dimension_semantics=("parallel",)),
    )(h3, hp3, sr, gamma.reshape(1, ls), beta.reshape(1, ls), w_lin,
      b_lin.reshape(ls, 1), mask3)


def kernel(logits, padding_mask, W_conv, gamma, beta, W_lin, b_lin):
    b, t, i_dim = logits.shape
    ls = W_conv.shape[0]
    m = b * t

    x = logits.reshape(m, i_dim)
    h, sr = _encode_and_stats(x, W_conv)

    with jax.ensure_compile_time_eval():
        perm = jax.random.permutation(
            jax.random.fold_in(jax.random.key(42), 0), t - 1)
        perm_full = jnp.concatenate(
            [perm.astype(jnp.int32), jnp.array([t - 1], jnp.int32)])
        idx = (jnp.arange(b, dtype=jnp.int32)[:, None] * t
               + perm_full[None, :]).reshape(-1)
        idx3 = idx.reshape(32, -1, 128)

    hp = _sc_gather(h, idx3)

    out0, out1, loss = _score(
        h.reshape(b, t, ls // 2), hp.reshape(b, t, ls // 2), sr, gamma, beta,
        W_lin, b_lin,
        padding_mask.astype(jnp.float32).reshape(b, 1, t), b, t)

    out = jnp.stack(
        [out0.reshape(b, t)[:, :t - 1], out1.reshape(b, t)[:, :t - 1]],
        axis=-1)
    return (out, loss.reshape(b, t)[:, :t - 1])
